# baseline (speedup 1.0000x reference)
import jax
import jax.numpy as jnp
from jax.experimental import pallas as pl
from jax.experimental.pallas import tpu as pltpu

N = 4096
NFEAT = 256
OUT = 128
BR = 64


def _gcn_kernel(w3_ref, w9_ref, f_ref, w3w_ref, w1w_ref, b3_ref, b1_ref,
                a_ref, at_ref, out_ref):
    m1 = (w3_ref[0, 0] * a_ref[0]
          + w3_ref[1, 0] * a_ref[1]
          + w3_ref[2, 0] * a_ref[2])
    p1 = jnp.dot(m1, f_ref[...], preferred_element_type=jnp.float32)
    u1 = jnp.dot(p1, w3w_ref[...], preferred_element_type=jnp.float32)
    out_ref[:, :OUT] = u1 + b3_ref[...]
    t = [w9_ref[k, 0] * at_ref[k] for k in range(9)]
    m2 = (((t[0] + t[1]) + (t[2] + t[3]))
          + ((t[4] + t[5]) + (t[6] + t[7]))) + t[8]
    p2 = jnp.dot(m2, f_ref[...], preferred_element_type=jnp.float32)
    u2 = jnp.dot(p2, w1w_ref[...], preferred_element_type=jnp.float32)
    out_ref[:, OUT:] = u2 + b1_ref[...]


def kernel(feature, A, A_t, W1, b1, W3, b3, weight_b, weight_b2):
    out = pl.pallas_call(
        _gcn_kernel,
        grid=(N // BR,),
        in_specs=[
            pl.BlockSpec(memory_space=pltpu.SMEM),
            pl.BlockSpec(memory_space=pltpu.SMEM),
            pl.BlockSpec((N, NFEAT), lambda i: (0, 0)),
            pl.BlockSpec((NFEAT, OUT), lambda i: (0, 0)),
            pl.BlockSpec((NFEAT, OUT), lambda i: (0, 0)),
            pl.BlockSpec((1, OUT), lambda i: (0, 0)),
            pl.BlockSpec((1, OUT), lambda i: (0, 0)),
            pl.BlockSpec((3, BR, N), lambda i: (0, i, 0)),
            pl.BlockSpec((9, BR, N), lambda i: (0, i, 0)),
        ],
        out_specs=pl.BlockSpec((BR, 2 * OUT), lambda i: (i, 0)),
        out_shape=jax.ShapeDtypeStruct((N, 2 * OUT), jnp.float32),
    )(weight_b2, weight_b, feature, W3, W1,
      b3.reshape(1, OUT), b1.reshape(1, OUT), A, A_t)
    return out


# speedup vs baseline: 1.0029x; 1.0029x over previous
import jax
import jax.numpy as jnp
from jax.experimental import pallas as pl
from jax.experimental.pallas import tpu as pltpu

N = 4096
NFEAT = 256
OUT = 128
BR = 64


def _gcn_kernel(w3_ref, w9_ref, f_ref, w3w_ref, w1w_ref, b3_ref, b1_ref,
                a_ref, at_ref, out_ref):
    m1 = (w3_ref[0, 0] * a_ref[0]
          + w3_ref[1, 0] * a_ref[1]
          + w3_ref[2, 0] * a_ref[2])
    p1 = jnp.dot(m1, f_ref[...], preferred_element_type=jnp.float32)
    u1 = jnp.dot(p1, w3w_ref[...], preferred_element_type=jnp.float32)
    out_ref[:, :OUT] = u1 + b3_ref[...]
    m2 = w9_ref[0, 0] * at_ref[0]
    for k in range(1, 9):
        m2 = m2 + w9_ref[k, 0] * at_ref[k]
    p2 = jnp.dot(m2, f_ref[...], preferred_element_type=jnp.float32)
    u2 = jnp.dot(p2, w1w_ref[...], preferred_element_type=jnp.float32)
    out_ref[:, OUT:] = u2 + b1_ref[...]


def kernel(feature, A, A_t, W1, b1, W3, b3, weight_b, weight_b2):
    out = pl.pallas_call(
        _gcn_kernel,
        grid=(N // BR,),
        in_specs=[
            pl.BlockSpec(memory_space=pltpu.SMEM),
            pl.BlockSpec(memory_space=pltpu.SMEM),
            pl.BlockSpec((N, NFEAT), lambda i: (0, 0)),
            pl.BlockSpec((NFEAT, OUT), lambda i: (0, 0)),
            pl.BlockSpec((NFEAT, OUT), lambda i: (0, 0)),
            pl.BlockSpec((1, OUT), lambda i: (0, 0)),
            pl.BlockSpec((1, OUT), lambda i: (0, 0)),
            pl.BlockSpec((3, BR, N), lambda i: (0, i, 0)),
            pl.BlockSpec((9, BR, N), lambda i: (0, i, 0)),
        ],
        out_specs=pl.BlockSpec((BR, 2 * OUT), lambda i: (i, 0)),
        out_shape=jax.ShapeDtypeStruct((N, 2 * OUT), jnp.float32),
    )(weight_b2, weight_b, feature, W3, W1,
      b3.reshape(1, OUT), b1.reshape(1, OUT), A, A_t)
    return out
